# baseline (speedup 1.0000x reference)
import functools

import jax
import jax.numpy as jnp
from jax import lax
from jax.experimental import pallas as pl
from jax.experimental.pallas import tpu as pltpu
from jax.experimental.pallas import tpu_sc as plsc

D_MODEL = 1024
MAX_LEN = 2048
BATCH = 4
LANES = 16
N_WORKERS = 32
B_FLAT = BATCH * MAX_LEN
B_PER_W = B_FLAT // N_WORKERS
CHUNK = 16
N_CHUNKS = B_PER_W // CHUNK
SCALE = 32.0

_mesh = plsc.VectorSubcoreMesh(core_axis_name="c", subcore_axis_name="s")


@functools.partial(
    pl.kernel,
    mesh=_mesh,
    out_type=jax.ShapeDtypeStruct((B_FLAT, D_MODEL), jnp.float32),
    scratch_types=[
        pltpu.VMEM((B_PER_W,), jnp.int32),
        pltpu.VMEM((CHUNK, D_MODEL), jnp.float32),
        pltpu.VMEM((CHUNK, D_MODEL), jnp.float32),
        pltpu.SemaphoreType.DMA,
    ],
)
def _emb_pe_kernel(x_hbm, table_hbm, pe_hbm, out_hbm, idx_v, rows_v, pe_v, sem):
    cid = lax.axis_index("c")
    sid = lax.axis_index("s")
    wid = sid * 2 + cid
    base = wid * B_PER_W
    p0 = lax.rem(base, MAX_LEN)

    pltpu.sync_copy(x_hbm.at[pl.ds(base, B_PER_W)], idx_v)

    def chunk_body(g, carry):
        off = g * CHUNK
        gather = pltpu.async_copy(
            table_hbm.at[idx_v.at[pl.ds(off, CHUNK)]], rows_v, sem)
        pltpu.sync_copy(pe_hbm.at[pl.ds(p0 + off, CHUNK)], pe_v)
        gather.wait()

        def row_body(j, c2):
            for l in range(D_MODEL // LANES):
                sl = pl.ds(l * LANES, LANES)
                rows_v[j, sl] = rows_v[j, sl] * SCALE + pe_v[j, sl]
            return c2

        lax.fori_loop(0, CHUNK, row_body, 0)
        pltpu.sync_copy(rows_v, out_hbm.at[pl.ds(base + off, CHUNK)])
        return carry

    lax.fori_loop(0, N_CHUNKS, chunk_body, 0)


def kernel(x, table, pe):
    xf = x.reshape(B_FLAT).astype(jnp.int32)
    pef = pe.reshape(MAX_LEN, D_MODEL)
    out = _emb_pe_kernel(xf, table, pef)
    return out.reshape(BATCH, MAX_LEN, D_MODEL)


# speedup vs baseline: 1.1894x; 1.1894x over previous
import functools

import jax
import jax.numpy as jnp
from jax import lax
from jax.experimental import pallas as pl
from jax.experimental.pallas import tpu as pltpu
from jax.experimental.pallas import tpu_sc as plsc

D_MODEL = 1024
MAX_LEN = 2048
BATCH = 4
LANES = 16
N_WORKERS = 32
B_FLAT = BATCH * MAX_LEN
B_PER_W = B_FLAT // N_WORKERS
CHUNK = 16
N_CHUNKS = B_PER_W // CHUNK
N_RB = 4
N_PB = 2
SCALE = 32.0

_mesh = plsc.VectorSubcoreMesh(core_axis_name="c", subcore_axis_name="s")


@functools.partial(
    pl.kernel,
    mesh=_mesh,
    out_type=jax.ShapeDtypeStruct((B_FLAT, D_MODEL), jnp.float32),
    scratch_types=[
        pltpu.VMEM((B_PER_W,), jnp.int32),
        pltpu.VMEM((N_RB, CHUNK, D_MODEL), jnp.float32),
        pltpu.VMEM((N_PB, CHUNK, D_MODEL), jnp.float32),
        [pltpu.SemaphoreType.DMA] * N_RB,
        [pltpu.SemaphoreType.DMA] * N_PB,
        [pltpu.SemaphoreType.DMA] * N_RB,
    ],
)
def _emb_pe_kernel(x_hbm, table_hbm, pe_hbm, out_hbm,
                   idx_v, rows_v, pe_v, sem_g, sem_p, sem_o):
    cid = lax.axis_index("c")
    sid = lax.axis_index("s")
    wid = sid * 2 + cid
    base = pl.multiple_of(wid * B_PER_W, B_PER_W)
    p0 = lax.rem(base, MAX_LEN)

    pltpu.sync_copy(x_hbm.at[pl.ds(base, B_PER_W)], idx_v)

    def issue_gather(c, b):
        off = pl.multiple_of(c * CHUNK, CHUNK)
        pltpu.async_copy(table_hbm.at[idx_v.at[pl.ds(off, CHUNK)]],
                         rows_v.at[b], sem_g[b])

    def issue_pe(c, b):
        off = pl.multiple_of(p0 + c * CHUNK, CHUNK)
        pltpu.async_copy(pe_hbm.at[pl.ds(off, CHUNK)], pe_v.at[b], sem_p[b])

    def issue_out(c, b):
        off = pl.multiple_of(base + c * CHUNK, CHUNK)
        pltpu.async_copy(rows_v.at[b], out_hbm.at[pl.ds(off, CHUNK)], sem_o[b])

    def wait_gather(b):
        pltpu.make_async_copy(table_hbm.at[pl.ds(0, CHUNK)],
                              rows_v.at[b], sem_g[b]).wait()

    def wait_pe(b):
        pltpu.make_async_copy(pe_hbm.at[pl.ds(0, CHUNK)],
                              pe_v.at[b], sem_p[b]).wait()

    def wait_out(b):
        pltpu.make_async_copy(rows_v.at[b], out_hbm.at[pl.ds(0, CHUNK)],
                              sem_o[b]).wait()

    issue_gather(0, 0)
    issue_gather(1, 1)
    issue_pe(0, 0)

    def quad_body(i, carry):
        for b in range(N_RB):
            c = i * N_RB + b
            pb = b % N_PB

            @pl.when((c >= 2) & (c < N_CHUNKS - 2))
            def _():
                wait_out((b + 2) % N_RB)

            @pl.when(c < N_CHUNKS - 2)
            def _():
                issue_gather(c + 2, (b + 2) % N_RB)

            @pl.when(c < N_CHUNKS - 1)
            def _():
                issue_pe(c + 1, (pb + 1) % N_PB)

            wait_gather(b)
            wait_pe(pb)

            def row_body(j, c2):
                for l in range(D_MODEL // LANES):
                    sl = pl.ds(l * LANES, LANES)
                    rows_v[b, j, sl] = rows_v[b, j, sl] * SCALE + pe_v[pb, j, sl]
                return c2

            lax.fori_loop(0, CHUNK, row_body, 0, unroll=False)
            issue_out(c, b)
        return carry

    lax.fori_loop(0, N_CHUNKS // N_RB, quad_body, 0, unroll=False)

    for b in range(N_RB):
        wait_out(b)


def kernel(x, table, pe):
    xf = x.reshape(B_FLAT).astype(jnp.int32)
    pef = pe.reshape(MAX_LEN, D_MODEL)
    out = _emb_pe_kernel(xf, table, pef)
    return out.reshape(BATCH, MAX_LEN, D_MODEL)
